# Initial kernel scaffold; baseline (speedup 1.0000x reference)
import jax
import jax.numpy as jnp
from jax.experimental import pallas as pl


def kernel(predict_pc, gt_pc):
    raise NotImplementedError("write your pallas kernel here")


# speedup vs baseline: 3.2402x; 3.2402x over previous
import jax
import jax.numpy as jnp
from jax.experimental import pallas as pl

_B, _N, _D = 8, 2048, 3


def _chamfer_body(a_ref, bt_ref, out_ref):
    b = pl.program_id(0)
    a = a_ref[0]
    bt = bt_ref[0]

    d = None
    for k in range(_D):
        ak = a[:, k:k + 1]
        bk = bt[k:k + 1, :]
        t = ak - bk
        t = t * t
        d = t if d is None else d + t

    rmin = jnp.min(d, axis=1, keepdims=True)
    cmin = jnp.min(d, axis=0, keepdims=True)
    s = (jnp.sum(rmin, axis=(0, 1), keepdims=True)
         + jnp.sum(cmin, axis=(0, 1), keepdims=True))

    @pl.when(b == 0)
    def _():
        out_ref[:, :] = jnp.zeros_like(s)

    out_ref[:, :] += s


def kernel(predict_pc, gt_pc):
    gtt = jnp.transpose(gt_pc, (0, 2, 1))
    out = pl.pallas_call(
        _chamfer_body,
        grid=(_B,),
        in_specs=[
            pl.BlockSpec((1, _N, _D), lambda b: (b, 0, 0)),
            pl.BlockSpec((1, _D, _N), lambda b: (b, 0, 0)),
        ],
        out_specs=pl.BlockSpec((1, 1), lambda b: (0, 0)),
        out_shape=jax.ShapeDtypeStruct((1, 1), jnp.float32),
    )(predict_pc, gtt)
    return out[0, 0] / (2.0 * _B * _N)
